# baseline (device time: 222734 ns/iter reference)
import jax
import jax.numpy as jnp
from jax import lax
from jax.experimental import pallas as pl
from jax.experimental.pallas import tpu as pltpu

N_DEV = 8
SQ = 1024
SKV_LOCAL = 1024
HQ = 8
DH = 128
D = HQ * DH
SCALE = 0.08838834764831843
BLK = 64
PACK = SQ + HQ


def _body(x_ref, wq_ref, k_ref, v_ref, wo_ref, out_ref,
          comm_ref, send_sems, recv_sems):
    pos = lax.axis_index("i")
    left = lax.rem(pos + N_DEV - 1, N_DEV)
    right = lax.rem(pos + 1, N_DEV)

    barrier_sem = pltpu.get_barrier_semaphore()
    for nbr in (left, right):
        pl.semaphore_signal(barrier_sem, inc=1, device_id=(nbr,),
                            device_id_type=pl.DeviceIdType.MESH)
    pl.semaphore_wait(barrier_sem, 2)

    q = jnp.dot(x_ref[...], wq_ref[...],
                preferred_element_type=jnp.float32).astype(jnp.bfloat16)

    qb = lax.broadcasted_iota(jnp.int32, (SQ, SKV_LOCAL), 0) // BLK
    kb = (lax.broadcasted_iota(jnp.int32, (SQ, SKV_LOCAL), 1)
          + pos * SKV_LOCAL) // BLK
    mask = (qb == kb) | (kb == 0) | ((qb + kb) % 3 == 0)

    kf = k_ref[...]
    vf = v_ref[...]
    o_parts = []
    l_parts = []
    for h in range(HQ):
        sl = slice(h * DH, (h + 1) * DH)
        s = lax.dot_general(q[:, sl], kf[:, sl], (((1,), (1,)), ((), ())),
                            preferred_element_type=jnp.float32) * SCALE
        p = jnp.where(mask, jnp.exp(s), 0.0)
        l_parts.append(jnp.sum(p, axis=1, keepdims=True))
        o_parts.append(jnp.dot(p.astype(jnp.bfloat16), vf[:, sl],
                               preferred_element_type=jnp.float32))
    o_full = jnp.concatenate(o_parts, axis=1)
    l_cols = jnp.concatenate(l_parts, axis=1)
    partial = jnp.concatenate([o_full, l_cols.T], axis=0)

    comm_ref[0] = partial.astype(jnp.bfloat16)
    acc = partial

    for h in range(N_DEV - 1):
        rdma = pltpu.make_async_remote_copy(
            src_ref=comm_ref.at[h],
            dst_ref=comm_ref.at[h + 1],
            send_sem=send_sems.at[h],
            recv_sem=recv_sems.at[h],
            device_id=(right,),
            device_id_type=pl.DeviceIdType.MESH,
        )
        rdma.start()
        rdma.wait()
        acc = acc + comm_ref[h + 1].astype(jnp.float32)

    o_sum = acc[:SQ, :]
    l_sum = acc[SQ:, :].T
    ctx = jnp.concatenate(
        [o_sum[:, h * DH:(h + 1) * DH] / l_sum[:, h:h + 1]
         for h in range(HQ)], axis=1).astype(jnp.bfloat16)
    out_ref[0] = jnp.dot(ctx, wo_ref[...],
                         preferred_element_type=jnp.float32)


def kernel(x, Wq, K_ext, V_ext, Wo):
    xb = x[0].astype(jnp.bfloat16)
    wqb = Wq.astype(jnp.bfloat16)
    kb = K_ext[0].reshape(SKV_LOCAL, D).astype(jnp.bfloat16)
    vb = V_ext[0].reshape(SKV_LOCAL, D).astype(jnp.bfloat16)
    wob = Wo.astype(jnp.bfloat16)

    return pl.pallas_call(
        _body,
        out_shape=jax.ShapeDtypeStruct((1, SQ, D), jnp.float32),
        in_specs=[pl.BlockSpec(memory_space=pltpu.VMEM)] * 5,
        out_specs=pl.BlockSpec(memory_space=pltpu.VMEM),
        scratch_shapes=[
            pltpu.VMEM((N_DEV, PACK, D), jnp.bfloat16),
            pltpu.SemaphoreType.DMA((N_DEV - 1,)),
            pltpu.SemaphoreType.DMA((N_DEV - 1,)),
        ],
        compiler_params=pltpu.CompilerParams(
            collective_id=0,
            vmem_limit_bytes=100 * 1024 * 1024,
        ),
    )(xb, wqb, kb, vb, wob)


# device time: 95907 ns/iter; 2.3224x vs baseline; 2.3224x over previous
import jax
import jax.numpy as jnp
from jax import lax
from jax.experimental import pallas as pl
from jax.experimental.pallas import tpu as pltpu

N_DEV = 8
SQ = 1024
SKV_LOCAL = 1024
HQ = 8
DH = 128
D = HQ * DH
SCALE = 0.08838834764831843
BLK = 64


def _body(x_ref, wq_ref, k_ref, v_ref, wo_ref, out_ref,
          stage_ref, lstage_ref, comm_ref, lcomm_ref, ctx_ref,
          send1, recv1, sendl, recvl, send2, recv2):
    pos = lax.axis_index("i")

    barrier_sem = pltpu.get_barrier_semaphore()
    for off in range(1, N_DEV):
        pl.semaphore_signal(barrier_sem, inc=1,
                            device_id=(lax.rem(pos + off, N_DEV),),
                            device_id_type=pl.DeviceIdType.MESH)
    pl.semaphore_wait(barrier_sem, N_DEV - 1)

    q = jnp.dot(x_ref[...], wq_ref[...],
                preferred_element_type=jnp.float32).astype(jnp.bfloat16)

    qb = lax.broadcasted_iota(jnp.int32, (SQ, SKV_LOCAL), 0) // BLK
    kb = (lax.broadcasted_iota(jnp.int32, (SQ, SKV_LOCAL), 1)
          + pos * SKV_LOCAL) // BLK
    mask = (qb == kb) | (kb == 0) | ((qb + kb) % 3 == 0)

    kf = k_ref[...]
    vf = v_ref[...]

    def p1_send(h):
        return pltpu.make_async_remote_copy(
            src_ref=stage_ref.at[:, pl.ds(h * DH, DH)],
            dst_ref=comm_ref.at[pos],
            send_sem=send1.at[h],
            recv_sem=recv1.at[pos],
            device_id=(h,),
            device_id_type=pl.DeviceIdType.MESH,
        )

    def l_send(off):
        return pltpu.make_async_remote_copy(
            src_ref=lstage_ref,
            dst_ref=lcomm_ref.at[pos],
            send_sem=sendl.at[off],
            recv_sem=recvl.at[pos],
            device_id=(lax.rem(pos + off, N_DEV),),
            device_id_type=pl.DeviceIdType.MESH,
        )

    l_parts = []
    for h in range(HQ):
        sl = slice(h * DH, (h + 1) * DH)
        s = lax.dot_general(q[:, sl], kf[:, sl], (((1,), (1,)), ((), ())),
                            preferred_element_type=jnp.float32) * SCALE
        p = jnp.where(mask, jnp.exp(s), 0.0)
        l_parts.append(jnp.sum(p, axis=1, keepdims=True))
        o_h = jnp.dot(p.astype(jnp.bfloat16), vf[:, sl],
                      preferred_element_type=jnp.float32)
        stage_ref[:, sl] = o_h.astype(jnp.bfloat16)

        @pl.when(pos != h)
        def _():
            p1_send(h).start()

    l_local = jnp.concatenate(l_parts, axis=1)
    lstage_ref[...] = l_local.astype(jnp.bfloat16)
    for off in range(1, N_DEV):
        l_send(off).start()

    acc = stage_ref[:, pl.ds(pos * DH, DH)].astype(jnp.float32)
    for off in range(1, N_DEV):
        src = lax.rem(pos + off, N_DEV)
        rdma = pltpu.make_async_remote_copy(
            src_ref=comm_ref.at[src],
            dst_ref=comm_ref.at[src],
            send_sem=send1.at[0],
            recv_sem=recv1.at[src],
            device_id=(pos,),
            device_id_type=pl.DeviceIdType.MESH,
        )
        rdma.wait_recv()
        acc = acc + comm_ref[src].astype(jnp.float32)

    l_sum = l_local
    for off in range(1, N_DEV):
        src = lax.rem(pos + off, N_DEV)
        rdma = pltpu.make_async_remote_copy(
            src_ref=lcomm_ref.at[src],
            dst_ref=lcomm_ref.at[src],
            send_sem=sendl.at[0],
            recv_sem=recvl.at[src],
            device_id=(pos,),
            device_id_type=pl.DeviceIdType.MESH,
        )
        rdma.wait_recv()
        l_sum = l_sum + lcomm_ref[src].astype(jnp.float32)

    onehot = (lax.broadcasted_iota(jnp.int32, (HQ, 1), 0) == pos
              ).astype(jnp.float32)
    l_col = jnp.dot(l_sum, onehot,
                    preferred_element_type=jnp.float32)
    ctx_mine = (acc / l_col).astype(jnp.bfloat16)
    stage_ref[:, pl.ds(pos * DH, DH)] = ctx_mine

    def p2_send(off):
        return pltpu.make_async_remote_copy(
            src_ref=stage_ref.at[:, pl.ds(pos * DH, DH)],
            dst_ref=ctx_ref.at[pos],
            send_sem=send2.at[off],
            recv_sem=recv2.at[pos],
            device_id=(lax.rem(pos + off, N_DEV),),
            device_id_type=pl.DeviceIdType.MESH,
        )

    for off in range(1, N_DEV):
        p2_send(off).start()

    out = jnp.dot(ctx_mine, wo_ref[pl.ds(pos * DH, DH), :],
                  preferred_element_type=jnp.float32)
    for off in range(1, N_DEV):
        src = lax.rem(pos + off, N_DEV)
        rdma = pltpu.make_async_remote_copy(
            src_ref=ctx_ref.at[src],
            dst_ref=ctx_ref.at[src],
            send_sem=send2.at[0],
            recv_sem=recv2.at[src],
            device_id=(pos,),
            device_id_type=pl.DeviceIdType.MESH,
        )
        rdma.wait_recv()
        out = out + jnp.dot(ctx_ref[src], wo_ref[pl.ds(src * DH, DH), :],
                            preferred_element_type=jnp.float32)
    out_ref[0] = out

    for h in range(HQ):
        @pl.when(pos != h)
        def _():
            p1_send(h).wait_send()
    for off in range(1, N_DEV):
        l_send(off).wait_send()
        p2_send(off).wait_send()


def kernel(x, Wq, K_ext, V_ext, Wo):
    xb = x[0].astype(jnp.bfloat16)
    wqb = Wq.astype(jnp.bfloat16)
    kb = K_ext[0].reshape(SKV_LOCAL, D).astype(jnp.bfloat16)
    vb = V_ext[0].reshape(SKV_LOCAL, D).astype(jnp.bfloat16)
    wob = Wo.astype(jnp.bfloat16)

    return pl.pallas_call(
        _body,
        out_shape=jax.ShapeDtypeStruct((1, SQ, D), jnp.float32),
        in_specs=[pl.BlockSpec(memory_space=pltpu.VMEM)] * 5,
        out_specs=pl.BlockSpec(memory_space=pltpu.VMEM),
        scratch_shapes=[
            pltpu.VMEM((SQ, D), jnp.bfloat16),
            pltpu.VMEM((SQ, HQ), jnp.bfloat16),
            pltpu.VMEM((N_DEV, SQ, DH), jnp.bfloat16),
            pltpu.VMEM((N_DEV, SQ, HQ), jnp.bfloat16),
            pltpu.VMEM((N_DEV, SQ, DH), jnp.bfloat16),
            pltpu.SemaphoreType.DMA((N_DEV,)),
            pltpu.SemaphoreType.DMA((N_DEV,)),
            pltpu.SemaphoreType.DMA((N_DEV,)),
            pltpu.SemaphoreType.DMA((N_DEV,)),
            pltpu.SemaphoreType.DMA((N_DEV,)),
            pltpu.SemaphoreType.DMA((N_DEV,)),
        ],
        compiler_params=pltpu.CompilerParams(
            collective_id=0,
            vmem_limit_bytes=100 * 1024 * 1024,
        ),
    )(xb, wqb, kb, vb, wob)


# device time: 79268 ns/iter; 2.8099x vs baseline; 1.2099x over previous
import jax
import jax.numpy as jnp
from jax import lax
from jax.experimental import pallas as pl
from jax.experimental.pallas import tpu as pltpu

N_DEV = 8
SQ = 1024
SKV_LOCAL = 1024
HQ = 8
DH = 128
D = HQ * DH
SCALE = 0.08838834764831843
BLK = 64


def _body(x_ref, wq_ref, k_ref, v_ref, wo_ref, out_ref,
          q3_ref, k3_ref, v3_ref, stage_ref, lstage_ref,
          comm_ref, lcomm_ref, ctx_ref,
          send1, recv1, sendl, recvl, send2, recv2):
    pos = lax.axis_index("i")

    barrier_sem = pltpu.get_barrier_semaphore()
    for off in range(1, N_DEV):
        pl.semaphore_signal(barrier_sem, inc=1,
                            device_id=(lax.rem(pos + off, N_DEV),),
                            device_id_type=pl.DeviceIdType.MESH)
    pl.semaphore_wait(barrier_sem, N_DEV - 1)

    q = jnp.dot(x_ref[...].astype(jnp.bfloat16),
                wq_ref[...].astype(jnp.bfloat16),
                preferred_element_type=jnp.float32).astype(jnp.bfloat16)
    for h in range(HQ):
        sl = slice(h * DH, (h + 1) * DH)
        q3_ref[h] = q[:, sl]
        k3_ref[h] = k_ref[:, sl].astype(jnp.bfloat16)
        v3_ref[h] = v_ref[:, sl].astype(jnp.bfloat16)

    qb = lax.broadcasted_iota(jnp.int32, (SQ, SKV_LOCAL), 0) // BLK
    kb = (lax.broadcasted_iota(jnp.int32, (SQ, SKV_LOCAL), 1)
          + pos * SKV_LOCAL) // BLK
    mask = (qb == kb) | (kb == 0) | ((qb + kb) % 3 == 0)

    def p1_send(idx, hd):
        return pltpu.make_async_remote_copy(
            src_ref=stage_ref.at[hd],
            dst_ref=comm_ref.at[pos],
            send_sem=send1.at[idx],
            recv_sem=recv1.at[pos],
            device_id=(hd,),
            device_id_type=pl.DeviceIdType.MESH,
        )

    def head_partial(hd):
        s = lax.dot_general(q3_ref[hd], k3_ref[hd],
                            (((1,), (1,)), ((), ())),
                            preferred_element_type=jnp.float32) * SCALE
        p = jnp.where(mask, jnp.exp(s), 0.0)
        l_h = jnp.sum(p, axis=1, keepdims=True)
        o_h = jnp.dot(p.astype(jnp.bfloat16), v3_ref[hd],
                      preferred_element_type=jnp.float32)
        return o_h, l_h

    l_parts = []
    for idx in range(N_DEV - 1):
        hd = lax.rem(pos + 1 + idx, N_DEV)
        o_h, l_h = head_partial(hd)
        l_parts.append(l_h)
        stage_ref[hd] = o_h.astype(jnp.bfloat16)
        p1_send(idx, hd).start()
    o_own, l_own = head_partial(pos)

    lstage_ref[...] = jnp.concatenate(l_parts + [l_own],
                                      axis=1).astype(jnp.bfloat16)

    def l_send(off):
        return pltpu.make_async_remote_copy(
            src_ref=lstage_ref,
            dst_ref=lcomm_ref.at[pos],
            send_sem=sendl.at[off],
            recv_sem=recvl.at[pos],
            device_id=(lax.rem(pos + off, N_DEV),),
            device_id_type=pl.DeviceIdType.MESH,
        )

    for off in range(1, N_DEV):
        l_send(off).start()

    acc = o_own
    l_col = l_own
    for k in range(N_DEV - 1):
        src = lax.rem(pos + (N_DEV - 1) - k, N_DEV)
        rdma = pltpu.make_async_remote_copy(
            src_ref=comm_ref.at[src],
            dst_ref=comm_ref.at[src],
            send_sem=send1.at[0],
            recv_sem=recv1.at[src],
            device_id=(pos,),
            device_id_type=pl.DeviceIdType.MESH,
        )
        rdma.wait_recv()
        acc = acc + comm_ref[src].astype(jnp.float32)
    for k in range(N_DEV - 1):
        src = lax.rem(pos + (N_DEV - 1) - k, N_DEV)
        rdma = pltpu.make_async_remote_copy(
            src_ref=lcomm_ref.at[src],
            dst_ref=lcomm_ref.at[src],
            send_sem=sendl.at[0],
            recv_sem=recvl.at[src],
            device_id=(pos,),
            device_id_type=pl.DeviceIdType.MESH,
        )
        rdma.wait_recv()
        l_col = l_col + lcomm_ref[src][:, k:k + 1].astype(jnp.float32)

    ctx_mine = (acc / l_col).astype(jnp.bfloat16)
    stage_ref[pos] = ctx_mine

    def p2_send(off):
        return pltpu.make_async_remote_copy(
            src_ref=stage_ref.at[pos],
            dst_ref=ctx_ref.at[pos],
            send_sem=send2.at[off],
            recv_sem=recv2.at[pos],
            device_id=(lax.rem(pos + off, N_DEV),),
            device_id_type=pl.DeviceIdType.MESH,
        )

    for off in range(1, N_DEV):
        p2_send(off).start()

    out = jnp.dot(ctx_mine,
                  wo_ref[pl.ds(pos * DH, DH), :].astype(jnp.bfloat16),
                  preferred_element_type=jnp.float32)
    for k in range(N_DEV - 1):
        src = lax.rem(pos + (N_DEV - 1) - k, N_DEV)
        rdma = pltpu.make_async_remote_copy(
            src_ref=ctx_ref.at[src],
            dst_ref=ctx_ref.at[src],
            send_sem=send2.at[0],
            recv_sem=recv2.at[src],
            device_id=(pos,),
            device_id_type=pl.DeviceIdType.MESH,
        )
        rdma.wait_recv()
        out = out + jnp.dot(
            ctx_ref[src],
            wo_ref[pl.ds(src * DH, DH), :].astype(jnp.bfloat16),
            preferred_element_type=jnp.float32)
    out_ref[0] = out

    for idx in range(N_DEV - 1):
        p1_send(idx, lax.rem(pos + 1 + idx, N_DEV)).wait_send()
    for off in range(1, N_DEV):
        l_send(off).wait_send()
        p2_send(off).wait_send()


def kernel(x, Wq, K_ext, V_ext, Wo):
    return pl.pallas_call(
        _body,
        out_shape=jax.ShapeDtypeStruct((1, SQ, D), jnp.float32),
        in_specs=[pl.BlockSpec(memory_space=pltpu.VMEM)] * 5,
        out_specs=pl.BlockSpec(memory_space=pltpu.VMEM),
        scratch_shapes=[
            pltpu.VMEM((HQ, SQ, DH), jnp.bfloat16),
            pltpu.VMEM((HQ, SKV_LOCAL, DH), jnp.bfloat16),
            pltpu.VMEM((HQ, SKV_LOCAL, DH), jnp.bfloat16),
            pltpu.VMEM((HQ, SQ, DH), jnp.bfloat16),
            pltpu.VMEM((SQ, HQ), jnp.bfloat16),
            pltpu.VMEM((N_DEV, SQ, DH), jnp.bfloat16),
            pltpu.VMEM((N_DEV, SQ, HQ), jnp.bfloat16),
            pltpu.VMEM((N_DEV, SQ, DH), jnp.bfloat16),
            pltpu.SemaphoreType.DMA((N_DEV,)),
            pltpu.SemaphoreType.DMA((N_DEV,)),
            pltpu.SemaphoreType.DMA((N_DEV,)),
            pltpu.SemaphoreType.DMA((N_DEV,)),
            pltpu.SemaphoreType.DMA((N_DEV,)),
            pltpu.SemaphoreType.DMA((N_DEV,)),
        ],
        compiler_params=pltpu.CompilerParams(
            collective_id=0,
            vmem_limit_bytes=100 * 1024 * 1024,
        ),
    )(x[0], Wq, K_ext[0].reshape(SKV_LOCAL, D), V_ext[0].reshape(SKV_LOCAL, D),
      Wo)
